# baseline (speedup 1.0000x reference)
import dataclasses
import functools

import jax
import jax.numpy as jnp
from jax import lax
from jax.experimental import pallas as pl
from jax.experimental.pallas import tpu as pltpu
from jax.experimental.pallas import tpu_sc as plsc

_N = 10000
_E = 320000
_D = 128
_NPAD = 10240

_NC = 2
_NS = 16
_NW = _NC * _NS

_EPAD = 327680
_HCHUNK = 2048
_EPW = _EPAD // _NW

_CHUNK = 128
_GPW = _EPAD // _CHUNK // _NW
_ISLOTS = 4
_RPS = _NPAD // _NS
_ZROWS = 32

_TCB = 1280
_TCG = _NPAD // _TCB


def _sc_mesh():
    return plsc.VectorSubcoreMesh(core_axis_name="c", subcore_axis_name="s")


def _sc_params():
    cp = pltpu.CompilerParams()
    if "needs_layout_passes" in pltpu.CompilerParams.__dataclass_fields__:
        cp = dataclasses.replace(cp, needs_layout_passes=False)
    return cp


def _hist_sc(col):
    @functools.partial(
        pl.kernel,
        out_type=jax.ShapeDtypeStruct((_NW, _NPAD), jnp.float32),
        mesh=_sc_mesh(),
        compiler_params=_sc_params(),
        scratch_types=[
            pltpu.VMEM((_HCHUNK,), jnp.int32),
            pltpu.VMEM((_NPAD,), jnp.float32),
        ],
    )
    def hist_kernel(col_hbm, out_hbm, idx_v, hist_v):
        cid = lax.axis_index("c")
        sid = lax.axis_index("s")
        wid = sid * _NC + cid

        zrow = jnp.zeros((16,), jnp.float32)
        ones16 = jnp.ones((16,), jnp.float32)

        @pl.loop(0, _NPAD, step=16)
        def _(i):
            hist_v[pl.ds(i, 16)] = zrow

        base = wid * _EPW

        @pl.loop(0, _EPW, step=_HCHUNK)
        def _(k):
            pltpu.sync_copy(col_hbm.at[pl.ds(base + k, _HCHUNK)], idx_v)

            @pl.loop(0, _HCHUNK, step=16)
            def _(j):
                idx16 = idx_v[pl.ds(j, 16)]
                plsc.addupdate_scatter(hist_v, [idx16], ones16)

        pltpu.sync_copy(hist_v, out_hbm.at[wid])

    return hist_kernel(col)


def _matmul_body(x_ref, w_ref, b_ref, xl_ref):
    xl_ref[...] = lax.dot_general(
        x_ref[...], w_ref[...], (((1,), (1,)), ((), ())),
        preferred_element_type=jnp.float32) + b_ref[...]


def _matmul_tc(x_pad, w2, b2_lin):
    return pl.pallas_call(
        _matmul_body,
        grid=(_TCG,),
        in_specs=[
            pl.BlockSpec((_TCB, _D), lambda i: (i, 0)),
            pl.BlockSpec((_D, _D), lambda i: (0, 0)),
            pl.BlockSpec((1, _D), lambda i: (0, 0)),
        ],
        out_specs=pl.BlockSpec((_TCB, _D), lambda i: (i, 0)),
        out_shape=jax.ShapeDtypeStruct((_NPAD, _D), jnp.float32),
    )(x_pad, w2, b2_lin.reshape(1, _D))


def _scale_body(xl_ref, dp_ref, y_ref):
    deg = 1.0 + jnp.sum(dp_ref[...], axis=0)
    dis = lax.rsqrt(deg)
    y_ref[...] = xl_ref[...] * dis[:, None]


def _scale_tc(xl, hist):
    return pl.pallas_call(
        _scale_body,
        grid=(_TCG,),
        in_specs=[
            pl.BlockSpec((_TCB, _D), lambda i: (i, 0)),
            pl.BlockSpec((_NW, _TCB), lambda i: (0, i)),
        ],
        out_specs=pl.BlockSpec((_TCB, _D), lambda i: (i, 0)),
        out_shape=jax.ShapeDtypeStruct((_NPAD, _D), jnp.float32),
    )(xl, hist)


def _agg_sc(rc2, y):
    @functools.partial(
        pl.kernel,
        out_type=jax.ShapeDtypeStruct((_NC, _NPAD, _D), jnp.float32),
        mesh=_sc_mesh(),
        scratch_types=[
            pltpu.VMEM((_ISLOTS, 2, _CHUNK), jnp.int32),
            pltpu.VMEM((_CHUNK, _D), jnp.float32),
            pltpu.VMEM((_CHUNK, _D), jnp.float32),
            pltpu.VMEM((_ZROWS, _D), jnp.float32),
            pltpu.VMEM_SHARED((_NPAD, _D), jnp.float32),
            pltpu.SemaphoreType.DMA((_ISLOTS,)),
            pltpu.SemaphoreType.DMA((2,)),
            pltpu.SemaphoreType.DMA((2,)),
        ],
    )
    def agg_kernel(rc_hbm, y_hbm, out_hbm,
                   idx_v, rows_v0, rows_v1, zero_v, agg_sh,
                   isem, gsem, ssem):
        cid = lax.axis_index("c")
        sid = lax.axis_index("s")
        wid = sid * _NC + cid
        base = wid * _GPW
        bufs = (rows_v0, rows_v1)

        zrow = jnp.zeros((16,), jnp.float32)

        @pl.loop(0, _ZROWS)
        def _(r):
            @pl.loop(0, _D, step=16)
            def _(j):
                zero_v[r, pl.ds(j, 16)] = zrow

        @pl.loop(0, _RPS, step=_ZROWS)
        def _(r):
            pltpu.sync_copy(zero_v, agg_sh.at[pl.ds(sid * _RPS + r, _ZROWS)])

        plsc.subcore_barrier()

        def idx_start(c):
            src = jnp.minimum(c, _GPW - 1)
            pltpu.async_copy(rc_hbm.at[base + src], idx_v.at[c % _ISLOTS],
                             isem.at[c % _ISLOTS])

        def idx_wait(c):
            src = jnp.minimum(c, _GPW - 1)
            pltpu.make_async_copy(
                rc_hbm.at[base + src], idx_v.at[c % _ISLOTS],
                isem.at[c % _ISLOTS]).wait()

        def gather_start(b, c):
            pltpu.async_copy(y_hbm.at[idx_v.at[c % _ISLOTS, 0]], bufs[b],
                             gsem.at[b])

        def gather_wait(b, c):
            pltpu.make_async_copy(
                y_hbm.at[idx_v.at[c % _ISLOTS, 0]], bufs[b],
                gsem.at[b]).wait()

        def scatter_start(b, c):
            pltpu.async_copy(
                bufs[b], agg_sh.at[idx_v.at[c % _ISLOTS, 1]], ssem.at[b],
                add=True)

        def scatter_wait(b, c):
            pltpu.make_async_copy(
                bufs[b], agg_sh.at[idx_v.at[c % _ISLOTS, 1]],
                ssem.at[b]).wait()

        idx_start(0)
        idx_start(1)
        idx_start(2)
        idx_start(3)
        idx_wait(0)
        gather_start(0, 0)
        idx_wait(1)
        gather_start(1, 1)

        @pl.loop(0, _GPW - 2, step=2)
        def _(g):
            gather_wait(0, g)
            scatter_start(0, g)
            gather_wait(1, g + 1)
            scatter_start(1, g + 1)
            idx_wait(g + 2)
            scatter_wait(0, g)
            gather_start(0, g + 2)
            idx_wait(g + 3)
            scatter_wait(1, g + 1)
            gather_start(1, g + 3)
            idx_start(g + 4)
            idx_start(g + 5)

        g_last = _GPW - 2
        gather_wait(0, g_last)
        scatter_start(0, g_last)
        gather_wait(1, g_last + 1)
        scatter_start(1, g_last + 1)
        scatter_wait(0, g_last)
        scatter_wait(1, g_last + 1)
        idx_wait(_GPW)
        idx_wait(_GPW + 1)

        plsc.subcore_barrier()
        pltpu.sync_copy(
            agg_sh.at[pl.ds(sid * _RPS, _RPS)],
            out_hbm.at[cid, pl.ds(sid * _RPS, _RPS)])

    return agg_kernel(rc2, y)


def _final_body(xl_ref, y_ref, agg_ref, dp_ref, m_ref, b_ref, out_ref):
    deg = 1.0 + jnp.sum(dp_ref[...], axis=0)
    dis = lax.rsqrt(deg)
    aggsum = agg_ref[0] + agg_ref[1] + y_ref[...]
    m = m_ref[...]
    out_ref[...] = (m * xl_ref[...] + (1.0 - m) * (aggsum * dis[:, None])
                    + b_ref[...])


def _final_tc(xl, y, agg_parts, hist, m_f32, bias):
    return pl.pallas_call(
        _final_body,
        grid=(_TCG,),
        in_specs=[
            pl.BlockSpec((_TCB, _D), lambda i: (i, 0)),
            pl.BlockSpec((_TCB, _D), lambda i: (i, 0)),
            pl.BlockSpec((_NC, _TCB, _D), lambda i: (0, i, 0)),
            pl.BlockSpec((_NW, _TCB), lambda i: (0, i)),
            pl.BlockSpec((_TCB, 1), lambda i: (i, 0)),
            pl.BlockSpec((1, _D), lambda i: (0, 0)),
        ],
        out_specs=pl.BlockSpec((_TCB, _D), lambda i: (i, 0)),
        out_shape=jax.ShapeDtypeStruct((_NPAD, _D), jnp.float32),
    )(xl, y, agg_parts, hist, m_f32, bias.reshape(1, _D))


def kernel(edge_index, edge_weight, x, M, W1, b1_lin, bias1, W2, b2_lin, bias2):
    row = edge_index[0]
    col = edge_index[1]
    pad_idx = _N + (jnp.arange(_EPAD - _E, dtype=jnp.int32) % (_NPAD - _N))
    row_p = jnp.concatenate([row, pad_idx])
    col_p = jnp.concatenate([col, pad_idx])
    rc2 = jnp.stack([row_p.reshape(_EPAD // _CHUNK, _CHUNK),
                     col_p.reshape(_EPAD // _CHUNK, _CHUNK)], axis=1)
    x_pad = jnp.concatenate(
        [x, jnp.zeros((_NPAD - _N, _D), jnp.float32)], axis=0)
    m_pad = jnp.concatenate(
        [M.astype(jnp.float32), jnp.zeros((_NPAD - _N, 1), jnp.float32)],
        axis=0)
    hist = _hist_sc(col_p)
    xl = _matmul_tc(x_pad, W2, b2_lin)
    y = _scale_tc(xl, hist)
    agg_parts = _agg_sc(rc2, y)
    out = _final_tc(xl, y, agg_parts, hist, m_pad, bias2)
    return out[:_N]


# speedup vs baseline: 1.0443x; 1.0443x over previous
import dataclasses
import functools

import jax
import jax.numpy as jnp
from jax import lax
from jax.experimental import pallas as pl
from jax.experimental.pallas import tpu as pltpu
from jax.experimental.pallas import tpu_sc as plsc

_N = 10000
_E = 320000
_D = 128
_NPAD = 10240

_NC = 2
_NS = 16
_NW = _NC * _NS

_EPW_H = _E // _NW
_HCHUNK = 2000

_CHUNK = 128
_NCH = _E // _CHUNK
_CPW = _NCH // _NW
_XTRA = _NCH - _CPW * _NW
_ISLOTS = 4
_RPS = _NPAD // _NS
_ZROWS = 32

_TCB = 1280
_TCG = _NPAD // _TCB


def _sc_mesh():
    return plsc.VectorSubcoreMesh(core_axis_name="c", subcore_axis_name="s")


def _sc_params():
    cp = pltpu.CompilerParams()
    if "needs_layout_passes" in pltpu.CompilerParams.__dataclass_fields__:
        cp = dataclasses.replace(cp, needs_layout_passes=False)
    return cp


def _hist_sc(col):
    @functools.partial(
        pl.kernel,
        out_type=jax.ShapeDtypeStruct((_NW, _NPAD), jnp.float32),
        mesh=_sc_mesh(),
        compiler_params=_sc_params(),
        scratch_types=[
            pltpu.VMEM((_HCHUNK,), jnp.int32),
            pltpu.VMEM((_NPAD,), jnp.float32),
        ],
    )
    def hist_kernel(col_hbm, out_hbm, idx_v, hist_v):
        cid = lax.axis_index("c")
        sid = lax.axis_index("s")
        wid = sid * _NC + cid

        zrow = jnp.zeros((16,), jnp.float32)
        ones16 = jnp.ones((16,), jnp.float32)

        @pl.loop(0, _NPAD, step=16)
        def _(i):
            hist_v[pl.ds(i, 16)] = zrow

        base = wid * _EPW_H

        @pl.loop(0, _EPW_H, step=_HCHUNK)
        def _(k):
            pltpu.sync_copy(col_hbm.at[pl.ds(base + k, _HCHUNK)], idx_v)

            @pl.loop(0, _HCHUNK, step=16)
            def _(j):
                idx16 = idx_v[pl.ds(j, 16)]
                plsc.addupdate_scatter(hist_v, [idx16], ones16)

        pltpu.sync_copy(hist_v, out_hbm.at[wid])

    return hist_kernel(col)


def _prep_body(x_ref, w_ref, b_ref, dp_ref, xl_ref, y_ref):
    xl = lax.dot_general(
        x_ref[...], w_ref[...], (((1,), (1,)), ((), ())),
        preferred_element_type=jnp.float32) + b_ref[...]
    deg = 1.0 + jnp.sum(dp_ref[...], axis=0)
    dis = lax.rsqrt(deg)
    xl_ref[...] = xl
    y_ref[...] = xl * dis[:, None]


def _prep_tc(x, w2, b2_lin, hist):
    return pl.pallas_call(
        _prep_body,
        grid=(_TCG,),
        in_specs=[
            pl.BlockSpec((_TCB, _D), lambda i: (i, 0)),
            pl.BlockSpec((_D, _D), lambda i: (0, 0)),
            pl.BlockSpec((1, _D), lambda i: (0, 0)),
            pl.BlockSpec((_NW, _TCB), lambda i: (0, i)),
        ],
        out_specs=[
            pl.BlockSpec((_TCB, _D), lambda i: (i, 0)),
            pl.BlockSpec((_TCB, _D), lambda i: (i, 0)),
        ],
        out_shape=[
            jax.ShapeDtypeStruct((_NPAD, _D), jnp.float32),
            jax.ShapeDtypeStruct((_NPAD, _D), jnp.float32),
        ],
    )(x, w2, b2_lin.reshape(1, _D), hist)


def _agg_sc(row1, col1, y):
    @functools.partial(
        pl.kernel,
        out_type=jax.ShapeDtypeStruct((_NC, _NPAD, _D), jnp.float32),
        mesh=_sc_mesh(),
        scratch_types=[
            pltpu.VMEM((_ISLOTS, 2, _CHUNK), jnp.int32),
            pltpu.VMEM((_CHUNK, _D), jnp.float32),
            pltpu.VMEM((_CHUNK, _D), jnp.float32),
            pltpu.VMEM((_ZROWS, _D), jnp.float32),
            pltpu.VMEM_SHARED((_NPAD, _D), jnp.float32),
            pltpu.SemaphoreType.DMA((_ISLOTS,)),
            pltpu.SemaphoreType.DMA((_ISLOTS,)),
            pltpu.SemaphoreType.DMA((2,)),
            pltpu.SemaphoreType.DMA((2,)),
        ],
    )
    def agg_kernel(row_hbm, col_hbm, y_hbm, out_hbm,
                   idx_v, rows_v0, rows_v1, zero_v, agg_sh,
                   irsem, icsem, gsem, ssem):
        cid = lax.axis_index("c")
        sid = lax.axis_index("s")
        wid = sid * _NC + cid
        base = wid * _CPW
        bufs = (rows_v0, rows_v1)

        zrow = jnp.zeros((16,), jnp.float32)

        @pl.loop(0, _ZROWS)
        def _(r):
            @pl.loop(0, _D, step=16)
            def _(j):
                zero_v[r, pl.ds(j, 16)] = zrow

        @pl.loop(0, _RPS, step=_ZROWS)
        def _(r):
            pltpu.sync_copy(zero_v, agg_sh.at[pl.ds(sid * _RPS + r, _ZROWS)])

        plsc.subcore_barrier()

        def idx_start(c):
            s = c % _ISLOTS
            g = base + jnp.minimum(c, _CPW - 1)
            pltpu.async_copy(row_hbm.at[pl.ds(g * _CHUNK, _CHUNK)],
                             idx_v.at[s, 0], irsem.at[s])
            pltpu.async_copy(col_hbm.at[pl.ds(g * _CHUNK, _CHUNK)],
                             idx_v.at[s, 1], icsem.at[s])

        def idx_wait(c):
            s = c % _ISLOTS
            g = base + jnp.minimum(c, _CPW - 1)
            pltpu.make_async_copy(row_hbm.at[pl.ds(g * _CHUNK, _CHUNK)],
                                  idx_v.at[s, 0], irsem.at[s]).wait()
            pltpu.make_async_copy(col_hbm.at[pl.ds(g * _CHUNK, _CHUNK)],
                                  idx_v.at[s, 1], icsem.at[s]).wait()

        def gather_start(b, c):
            pltpu.async_copy(y_hbm.at[idx_v.at[c % _ISLOTS, 0]], bufs[b],
                             gsem.at[b])

        def gather_wait(b, c):
            pltpu.make_async_copy(
                y_hbm.at[idx_v.at[c % _ISLOTS, 0]], bufs[b],
                gsem.at[b]).wait()

        def scatter_start(b, c):
            pltpu.async_copy(
                bufs[b], agg_sh.at[idx_v.at[c % _ISLOTS, 1]], ssem.at[b],
                add=True)

        def scatter_wait(b, c):
            pltpu.make_async_copy(
                bufs[b], agg_sh.at[idx_v.at[c % _ISLOTS, 1]],
                ssem.at[b]).wait()

        idx_start(0)
        idx_start(1)
        idx_start(2)
        idx_start(3)
        idx_wait(0)
        gather_start(0, 0)
        idx_wait(1)
        gather_start(1, 1)

        @pl.loop(0, _CPW - 2, step=2)
        def _(g):
            gather_wait(0, g)
            scatter_start(0, g)
            gather_wait(1, g + 1)
            scatter_start(1, g + 1)
            idx_wait(g + 2)
            scatter_wait(0, g)
            gather_start(0, g + 2)
            idx_wait(g + 3)
            scatter_wait(1, g + 1)
            gather_start(1, g + 3)
            idx_start(g + 4)
            idx_start(g + 5)

        g_last = _CPW - 2
        gather_wait(0, g_last)
        scatter_start(0, g_last)
        gather_wait(1, g_last + 1)
        scatter_start(1, g_last + 1)
        scatter_wait(0, g_last)
        scatter_wait(1, g_last + 1)
        idx_wait(_CPW)
        idx_wait(_CPW + 1)

        @pl.when(wid < _XTRA)
        def _():
            gx = _CPW * _NW + wid
            pltpu.sync_copy(row_hbm.at[pl.ds(gx * _CHUNK, _CHUNK)],
                            idx_v.at[0, 0])
            pltpu.sync_copy(col_hbm.at[pl.ds(gx * _CHUNK, _CHUNK)],
                            idx_v.at[0, 1])
            pltpu.async_copy(y_hbm.at[idx_v.at[0, 0]], rows_v0, gsem.at[0])
            pltpu.make_async_copy(
                y_hbm.at[idx_v.at[0, 0]], rows_v0, gsem.at[0]).wait()
            pltpu.async_copy(rows_v0, agg_sh.at[idx_v.at[0, 1]], ssem.at[0],
                             add=True)
            pltpu.make_async_copy(
                rows_v0, agg_sh.at[idx_v.at[0, 1]], ssem.at[0]).wait()

        plsc.subcore_barrier()
        pltpu.sync_copy(
            agg_sh.at[pl.ds(sid * _RPS, _RPS)],
            out_hbm.at[cid, pl.ds(sid * _RPS, _RPS)])

    return agg_kernel(row1, col1, y)


def _final_body(xl_ref, agg_ref, dp_ref, m_ref, b_ref, out_ref):
    deg = 1.0 + jnp.sum(dp_ref[...], axis=0)
    dis = lax.rsqrt(deg)
    xl = xl_ref[...]
    aggsum = agg_ref[0] + agg_ref[1] + xl * dis[:, None]
    m = m_ref[...]
    out_ref[...] = (m * xl + (1.0 - m) * (aggsum * dis[:, None])
                    + b_ref[...])


def _final_tc(xl, agg_parts, hist, m_f32, bias):
    return pl.pallas_call(
        _final_body,
        grid=(_TCG,),
        in_specs=[
            pl.BlockSpec((_TCB, _D), lambda i: (i, 0)),
            pl.BlockSpec((_NC, _TCB, _D), lambda i: (0, i, 0)),
            pl.BlockSpec((_NW, _TCB), lambda i: (0, i)),
            pl.BlockSpec((_TCB, 1), lambda i: (i, 0)),
            pl.BlockSpec((1, _D), lambda i: (0, 0)),
        ],
        out_specs=pl.BlockSpec((_TCB, _D), lambda i: (i, 0)),
        out_shape=jax.ShapeDtypeStruct((_N, _D), jnp.float32),
    )(xl, agg_parts, hist, m_f32, bias.reshape(1, _D))


def kernel(edge_index, edge_weight, x, M, W1, b1_lin, bias1, W2, b2_lin, bias2):
    row1 = edge_index[0]
    col1 = edge_index[1]
    m_f32 = M.astype(jnp.float32)
    hist = _hist_sc(col1)
    xl, y = _prep_tc(x, W2, b2_lin, hist)
    agg_parts = _agg_sc(row1, col1, y)
    return _final_tc(xl, agg_parts, hist, m_f32, bias2)


# speedup vs baseline: 1.0563x; 1.0115x over previous
import dataclasses
import functools

import jax
import jax.numpy as jnp
from jax import lax
from jax.experimental import pallas as pl
from jax.experimental.pallas import tpu as pltpu
from jax.experimental.pallas import tpu_sc as plsc

_N = 10000
_E = 320000
_D = 128
_NPAD = 10240

_NC = 2
_NS = 16
_NW = _NC * _NS

_EPW_H = _E // _NW
_HCHUNK = 2000
_NHCH = _EPW_H // _HCHUNK

_CHUNK = 128
_NCH = _E // _CHUNK
_CPW = _NCH // _NW
_XTRA = _NCH - _CPW * _NW
_ISLOTS = 4
_RPS = _NPAD // _NS
_ZROWS = 32

_TCB = 1280
_TCG = _NPAD // _TCB


def _sc_mesh():
    return plsc.VectorSubcoreMesh(core_axis_name="c", subcore_axis_name="s")


def _sc_params():
    cp = pltpu.CompilerParams()
    if "needs_layout_passes" in pltpu.CompilerParams.__dataclass_fields__:
        cp = dataclasses.replace(cp, needs_layout_passes=False)
    return cp


def _hist_sc(col):
    @functools.partial(
        pl.kernel,
        out_type=jax.ShapeDtypeStruct((_NW, _NPAD), jnp.float32),
        mesh=_sc_mesh(),
        compiler_params=_sc_params(),
        scratch_types=[
            pltpu.VMEM((_HCHUNK,), jnp.int32),
            pltpu.VMEM((_HCHUNK,), jnp.int32),
            pltpu.VMEM((_NPAD,), jnp.float32),
            pltpu.SemaphoreType.DMA((2,)),
        ],
    )
    def hist_kernel(col_hbm, out_hbm, idx_v0, idx_v1, hist_v, hsem):
        cid = lax.axis_index("c")
        sid = lax.axis_index("s")
        wid = sid * _NC + cid
        bufs = (idx_v0, idx_v1)

        zrow = jnp.zeros((16,), jnp.float32)
        ones16 = jnp.ones((16,), jnp.float32)

        @pl.loop(0, _NPAD, step=16)
        def _(i):
            hist_v[pl.ds(i, 16)] = zrow

        base = wid * _EPW_H

        def chunk_start(k):
            pltpu.async_copy(
                col_hbm.at[pl.ds(base + k * _HCHUNK, _HCHUNK)],
                bufs[k % 2], hsem.at[k % 2])

        def chunk_wait(k):
            pltpu.make_async_copy(
                col_hbm.at[pl.ds(base + k * _HCHUNK, _HCHUNK)],
                bufs[k % 2], hsem.at[k % 2]).wait()

        chunk_start(0)

        for k in range(_NHCH):
            chunk_wait(k)
            if k + 1 < _NHCH:
                chunk_start(k + 1)
            idx_b = bufs[k % 2]

            @pl.loop(0, _HCHUNK, step=80)
            def _(j, idx_b=idx_b):
                plsc.addupdate_scatter(
                    hist_v, [idx_b[pl.ds(j, 16)]], ones16)
                plsc.addupdate_scatter(
                    hist_v, [idx_b[pl.ds(j + 16, 16)]], ones16)
                plsc.addupdate_scatter(
                    hist_v, [idx_b[pl.ds(j + 32, 16)]], ones16)
                plsc.addupdate_scatter(
                    hist_v, [idx_b[pl.ds(j + 48, 16)]], ones16)
                plsc.addupdate_scatter(
                    hist_v, [idx_b[pl.ds(j + 64, 16)]], ones16)

        pltpu.sync_copy(hist_v, out_hbm.at[wid])

    return hist_kernel(col)


def _prep_body(x_ref, w_ref, b_ref, dp_ref, y_ref):
    xl = lax.dot_general(
        x_ref[...], w_ref[...], (((1,), (1,)), ((), ())),
        preferred_element_type=jnp.float32) + b_ref[...]
    deg = 1.0 + jnp.sum(dp_ref[...], axis=0)
    dis = lax.rsqrt(deg)
    y_ref[...] = xl * dis[:, None]


def _prep_tc(x, w2, b2_lin, hist):
    return pl.pallas_call(
        _prep_body,
        grid=(_TCG,),
        in_specs=[
            pl.BlockSpec((_TCB, _D), lambda i: (i, 0)),
            pl.BlockSpec((_D, _D), lambda i: (0, 0)),
            pl.BlockSpec((1, _D), lambda i: (0, 0)),
            pl.BlockSpec((_NW, _TCB), lambda i: (0, i)),
        ],
        out_specs=pl.BlockSpec((_TCB, _D), lambda i: (i, 0)),
        out_shape=jax.ShapeDtypeStruct((_NPAD, _D), jnp.float32),
    )(x, w2, b2_lin.reshape(1, _D), hist)


def _agg_sc(row1, col1, y):
    @functools.partial(
        pl.kernel,
        out_type=jax.ShapeDtypeStruct((_NC, _NPAD, _D), jnp.float32),
        mesh=_sc_mesh(),
        scratch_types=[
            pltpu.VMEM((_ISLOTS, 2, _CHUNK), jnp.int32),
            pltpu.VMEM((_CHUNK, _D), jnp.float32),
            pltpu.VMEM((_CHUNK, _D), jnp.float32),
            pltpu.VMEM((_ZROWS, _D), jnp.float32),
            pltpu.VMEM_SHARED((_NPAD, _D), jnp.float32),
            pltpu.SemaphoreType.DMA((_ISLOTS,)),
            pltpu.SemaphoreType.DMA((_ISLOTS,)),
            pltpu.SemaphoreType.DMA((2,)),
            pltpu.SemaphoreType.DMA((2,)),
        ],
    )
    def agg_kernel(row_hbm, col_hbm, y_hbm, out_hbm,
                   idx_v, rows_v0, rows_v1, zero_v, agg_sh,
                   irsem, icsem, gsem, ssem):
        cid = lax.axis_index("c")
        sid = lax.axis_index("s")
        wid = sid * _NC + cid
        base = wid * _CPW
        bufs = (rows_v0, rows_v1)

        zrow = jnp.zeros((16,), jnp.float32)

        @pl.loop(0, _ZROWS)
        def _(r):
            @pl.loop(0, _D, step=16)
            def _(j):
                zero_v[r, pl.ds(j, 16)] = zrow

        @pl.loop(0, _RPS, step=_ZROWS)
        def _(r):
            pltpu.sync_copy(zero_v, agg_sh.at[pl.ds(sid * _RPS + r, _ZROWS)])

        plsc.subcore_barrier()

        def idx_start(c):
            s = c % _ISLOTS
            g = base + jnp.minimum(c, _CPW - 1)
            pltpu.async_copy(row_hbm.at[pl.ds(g * _CHUNK, _CHUNK)],
                             idx_v.at[s, 0], irsem.at[s])
            pltpu.async_copy(col_hbm.at[pl.ds(g * _CHUNK, _CHUNK)],
                             idx_v.at[s, 1], icsem.at[s])

        def idx_wait(c):
            s = c % _ISLOTS
            g = base + jnp.minimum(c, _CPW - 1)
            pltpu.make_async_copy(row_hbm.at[pl.ds(g * _CHUNK, _CHUNK)],
                                  idx_v.at[s, 0], irsem.at[s]).wait()
            pltpu.make_async_copy(col_hbm.at[pl.ds(g * _CHUNK, _CHUNK)],
                                  idx_v.at[s, 1], icsem.at[s]).wait()

        def gather_start(b, c):
            pltpu.async_copy(y_hbm.at[idx_v.at[c % _ISLOTS, 0]], bufs[b],
                             gsem.at[b])

        def gather_wait(b, c):
            pltpu.make_async_copy(
                y_hbm.at[idx_v.at[c % _ISLOTS, 0]], bufs[b],
                gsem.at[b]).wait()

        def scatter_start(b, c):
            pltpu.async_copy(
                bufs[b], agg_sh.at[idx_v.at[c % _ISLOTS, 1]], ssem.at[b],
                add=True)

        def scatter_wait(b, c):
            pltpu.make_async_copy(
                bufs[b], agg_sh.at[idx_v.at[c % _ISLOTS, 1]],
                ssem.at[b]).wait()

        idx_start(0)
        idx_start(1)
        idx_start(2)
        idx_start(3)
        idx_wait(0)
        gather_start(0, 0)
        idx_wait(1)
        gather_start(1, 1)

        @pl.loop(0, _CPW - 2, step=2)
        def _(g):
            gather_wait(0, g)
            scatter_start(0, g)
            gather_wait(1, g + 1)
            scatter_start(1, g + 1)
            idx_wait(g + 2)
            scatter_wait(0, g)
            gather_start(0, g + 2)
            idx_wait(g + 3)
            scatter_wait(1, g + 1)
            gather_start(1, g + 3)
            idx_start(g + 4)
            idx_start(g + 5)

        g_last = _CPW - 2
        gather_wait(0, g_last)
        scatter_start(0, g_last)
        gather_wait(1, g_last + 1)
        scatter_start(1, g_last + 1)
        scatter_wait(0, g_last)
        scatter_wait(1, g_last + 1)
        idx_wait(_CPW)
        idx_wait(_CPW + 1)

        @pl.when(wid < _XTRA)
        def _():
            gx = _CPW * _NW + wid
            pltpu.sync_copy(row_hbm.at[pl.ds(gx * _CHUNK, _CHUNK)],
                            idx_v.at[0, 0])
            pltpu.sync_copy(col_hbm.at[pl.ds(gx * _CHUNK, _CHUNK)],
                            idx_v.at[0, 1])
            pltpu.async_copy(y_hbm.at[idx_v.at[0, 0]], rows_v0, gsem.at[0])
            pltpu.make_async_copy(
                y_hbm.at[idx_v.at[0, 0]], rows_v0, gsem.at[0]).wait()
            pltpu.async_copy(rows_v0, agg_sh.at[idx_v.at[0, 1]], ssem.at[0],
                             add=True)
            pltpu.make_async_copy(
                rows_v0, agg_sh.at[idx_v.at[0, 1]], ssem.at[0]).wait()

        plsc.subcore_barrier()
        pltpu.sync_copy(
            agg_sh.at[pl.ds(sid * _RPS, _RPS)],
            out_hbm.at[cid, pl.ds(sid * _RPS, _RPS)])

    return agg_kernel(row1, col1, y)


def _final_body(y_ref, agg_ref, dp_ref, m_ref, b_ref, out_ref):
    deg = 1.0 + jnp.sum(dp_ref[...], axis=0)
    dis = lax.rsqrt(deg)
    y = y_ref[...]
    xl = y * jnp.sqrt(deg)[:, None]
    aggsum = agg_ref[0] + agg_ref[1] + y
    m = m_ref[...]
    out_ref[...] = (m * xl + (1.0 - m) * (aggsum * dis[:, None])
                    + b_ref[...])


def _final_tc(y, agg_parts, hist, m_f32, bias):
    return pl.pallas_call(
        _final_body,
        grid=(_TCG,),
        in_specs=[
            pl.BlockSpec((_TCB, _D), lambda i: (i, 0)),
            pl.BlockSpec((_NC, _TCB, _D), lambda i: (0, i, 0)),
            pl.BlockSpec((_NW, _TCB), lambda i: (0, i)),
            pl.BlockSpec((_TCB, 1), lambda i: (i, 0)),
            pl.BlockSpec((1, _D), lambda i: (0, 0)),
        ],
        out_specs=pl.BlockSpec((_TCB, _D), lambda i: (i, 0)),
        out_shape=jax.ShapeDtypeStruct((_N, _D), jnp.float32),
    )(y, agg_parts, hist, m_f32, bias.reshape(1, _D))


def kernel(edge_index, edge_weight, x, M, W1, b1_lin, bias1, W2, b2_lin, bias2):
    row1 = edge_index[0]
    col1 = edge_index[1]
    m_f32 = M.astype(jnp.float32)
    hist = _hist_sc(col1)
    y = _prep_tc(x, W2, b2_lin, hist)
    agg_parts = _agg_sc(row1, col1, y)
    return _final_tc(y, agg_parts, hist, m_f32, bias2)
